# baseline (speedup 1.0000x reference)
import jax
import jax.numpy as jnp
from jax import lax
from jax.experimental import pallas as pl
from jax.experimental.pallas import tpu as pltpu

E = 8
K = 2
D = 768
DFF = 768
OUT = 768
TOK = 2048

BT = 512
NTT = TOK // BT


def _moe_body(x_ref, wg_ref, bg_ref, w1_ref, b1_ref, w2_ref, b2_ref, out_ref,
              w1fb_ref, w2b_ref, mwh_ref, mw_ref):
    t = pl.program_id(0)
    s = pl.program_id(1)

    @pl.when(s == 0)
    def _():
        x = x_ref[...]
        logits = jnp.dot(x, wg_ref[...], preferred_element_type=jnp.float32)
        logits = logits + bg_ref[0]
        lane = lax.broadcasted_iota(jnp.int32, (BT, E), 1)
        m1 = jnp.max(logits, axis=1, keepdims=True)
        i1 = jnp.min(jnp.where(logits == m1, lane, E), axis=1, keepdims=True)
        l2 = jnp.where(lane == i1, -jnp.inf, logits)
        m2 = jnp.max(l2, axis=1, keepdims=True)
        i2 = jnp.min(jnp.where(l2 == m2, lane, E), axis=1, keepdims=True)
        w_top = 1.0 / (1.0 + jnp.exp(m2 - m1))
        mw_ref[...] = jnp.where(lane == i1, w_top, 0.0) + jnp.where(
            lane == i2, 1.0 - w_top, 0.0)

    @pl.when((t == 0) & (s < E))
    def _():
        w1fb_ref[s] = (w1_ref[0, :D, :] + w1_ref[0, D:, :]).astype(
            jnp.bfloat16)
        w2b_ref[pl.ds(s * DFF, DFF), :] = w2_ref[0].astype(jnp.bfloat16)

    @pl.when(s < E)
    def _():
        lane = lax.broadcasted_iota(jnp.int32, (BT, E), 1)
        mwcol = jnp.sum(mw_ref[...] * (lane == s).astype(jnp.float32),
                        axis=1, keepdims=True)
        sub = lax.broadcasted_iota(jnp.int32, (E, DFF), 0)
        b1row = jnp.sum(b1_ref[...] * (sub == s).astype(jnp.float32),
                        axis=0, keepdims=True)
        xb = x_ref[...].astype(jnp.bfloat16)
        h = jnp.maximum(
            jnp.dot(xb, w1fb_ref[s], preferred_element_type=jnp.float32)
            + b1row, 0.0)
        mwh_ref[:, pl.ds(s * DFF, DFF)] = (mwcol * h).astype(jnp.bfloat16)

    @pl.when(s == E)
    def _():
        y = jnp.dot(mwh_ref[...], w2b_ref[...],
                    preferred_element_type=jnp.float32)
        yb = jnp.dot(mw_ref[...], b2_ref[...],
                     preferred_element_type=jnp.float32)
        out_ref[...] = y + yb


def kernel(inputs, Wg, bg, W1, b1, W2, b2):
    out = pl.pallas_call(
        _moe_body,
        grid=(NTT, E + 1),
        in_specs=[
            pl.BlockSpec((BT, D), lambda t, s: (t, 0)),
            pl.BlockSpec((D, E), lambda t, s: (0, 0)),
            pl.BlockSpec((1, E), lambda t, s: (0, 0)),
            pl.BlockSpec((1, 2 * D, DFF),
                         lambda t, s: (jnp.where(t == 0,
                                                 jnp.minimum(s, E - 1),
                                                 E - 1), 0, 0)),
            pl.BlockSpec((E, DFF), lambda t, s: (0, 0)),
            pl.BlockSpec((1, DFF, OUT),
                         lambda t, s: (jnp.where(t == 0,
                                                 jnp.minimum(s, E - 1),
                                                 E - 1), 0, 0)),
            pl.BlockSpec((E, OUT), lambda t, s: (0, 0)),
        ],
        out_specs=pl.BlockSpec((BT, OUT), lambda t, s: (t, 0)),
        out_shape=jax.ShapeDtypeStruct((TOK, OUT), jnp.float32),
        scratch_shapes=[
            pltpu.VMEM((E, D, DFF), jnp.bfloat16),
            pltpu.VMEM((E * DFF, OUT), jnp.bfloat16),
            pltpu.VMEM((BT, E * DFF), jnp.bfloat16),
            pltpu.VMEM((BT, E), jnp.float32),
        ],
        compiler_params=pltpu.CompilerParams(
            dimension_semantics=("arbitrary", "arbitrary"),
        ),
    )(inputs, Wg, bg.reshape(1, E), W1, b1, W2, b2)
    return out


kernel = jax.jit(kernel)


# speedup vs baseline: 1.0183x; 1.0183x over previous
import jax
import jax.numpy as jnp
from jax import lax
from jax.experimental import pallas as pl
from jax.experimental.pallas import tpu as pltpu

E = 8
K = 2
D = 768
DFF = 768
OUT = 768
TOK = 2048


def _moe_body(x_ref, wg_ref, bg_ref, w1_ref, b1_ref, w2_ref, b2_ref, out_ref,
              ha_ref, hb_ref, mw_ref):
    e = pl.program_id(0)

    @pl.when(e == 0)
    def _():
        x = x_ref[...]
        logits = jnp.dot(x, wg_ref[...], preferred_element_type=jnp.float32)
        logits = logits + bg_ref[0]
        lane = lax.broadcasted_iota(jnp.int32, (TOK, E), 1)
        m1 = jnp.max(logits, axis=1, keepdims=True)
        i1 = jnp.min(jnp.where(logits == m1, lane, E), axis=1, keepdims=True)
        l2 = jnp.where(lane == i1, -jnp.inf, logits)
        m2 = jnp.max(l2, axis=1, keepdims=True)
        i2 = jnp.min(jnp.where(l2 == m2, lane, E), axis=1, keepdims=True)
        t = jnp.exp(m2 - m1)
        wa = 1.0 / (1.0 + t)
        wb = 1.0 - wa
        mw_ref[...] = jnp.where(lane == i1, wa, 0.0) + jnp.where(
            lane == i2, wb, 0.0)

    @pl.when(e < E)
    def _():
        w1f = (w1_ref[0, :D, :] + w1_ref[0, D:, :]).astype(jnp.bfloat16)
        h = jnp.maximum(
            jnp.dot(x_ref[...].astype(jnp.bfloat16), w1f,
                    preferred_element_type=jnp.float32)
            + b1_ref[0, 0], 0.0).astype(jnp.bfloat16)

        @pl.when(e % 2 == 0)
        def _():
            ha_ref[...] = h

        @pl.when(e % 2 == 1)
        def _():
            hb_ref[...] = h

    @pl.when(e > 0)
    def _():
        ep = e - 1
        mw = jnp.sum(
            mw_ref[...]
            * (lax.broadcasted_iota(jnp.int32, (TOK, E), 1) == ep).astype(
                jnp.float32),
            axis=1, keepdims=True)
        w2b = w2_ref[0].astype(jnp.bfloat16)

        def consume(h_ref):
            y = jnp.dot(h_ref[...], w2b, preferred_element_type=jnp.float32)
            contrib = mw * (y + b2_ref[0, 0])

            @pl.when(ep == 0)
            def _():
                out_ref[...] = contrib

            @pl.when(ep > 0)
            def _():
                out_ref[...] += contrib

        @pl.when(ep % 2 == 0)
        def _():
            consume(ha_ref)

        @pl.when(ep % 2 == 1)
        def _():
            consume(hb_ref)


def kernel(inputs, Wg, bg, W1, b1, W2, b2):
    bg2 = bg.reshape(1, E)
    b1r = b1.reshape(E, 1, DFF)
    b2r = b2.reshape(E, 1, OUT)
    out = pl.pallas_call(
        _moe_body,
        grid=(E + 1,),
        in_specs=[
            pl.BlockSpec((TOK, D), lambda e: (0, 0)),
            pl.BlockSpec((D, E), lambda e: (0, 0)),
            pl.BlockSpec((1, E), lambda e: (0, 0)),
            pl.BlockSpec((1, 2 * D, DFF),
                         lambda e: (jnp.minimum(e, E - 1), 0, 0)),
            pl.BlockSpec((1, 1, DFF), lambda e: (jnp.minimum(e, E - 1), 0, 0)),
            pl.BlockSpec((1, DFF, OUT),
                         lambda e: (jnp.maximum(e - 1, 0), 0, 0)),
            pl.BlockSpec((1, 1, OUT), lambda e: (jnp.maximum(e - 1, 0), 0, 0)),
        ],
        out_specs=pl.BlockSpec((TOK, OUT), lambda e: (0, 0)),
        out_shape=jax.ShapeDtypeStruct((TOK, OUT), jnp.float32),
        scratch_shapes=[
            pltpu.VMEM((TOK, DFF), jnp.bfloat16),
            pltpu.VMEM((TOK, DFF), jnp.bfloat16),
            pltpu.VMEM((TOK, E), jnp.float32),
        ],
        compiler_params=pltpu.CompilerParams(
            dimension_semantics=("arbitrary",),
        ),
    )(inputs, Wg, bg2, W1, b1r, W2, b2r)
    return out


kernel = jax.jit(kernel)


# speedup vs baseline: 1.0222x; 1.0038x over previous
import jax
import jax.numpy as jnp
from jax import lax
from jax.experimental import pallas as pl
from jax.experimental.pallas import tpu as pltpu

E = 8
K = 2
D = 768
DFF = 768
OUT = 768
TOK = 2048


def _moe_body(x_ref, wg_ref, bg_ref, w1_ref, b1_ref, w2_ref, b2_ref, out_ref,
              ha_ref, hb_ref, mw_ref):
    e = pl.program_id(0)

    @pl.when(e == 0)
    def _():
        x = x_ref[...]
        logits = jnp.dot(x, wg_ref[...], preferred_element_type=jnp.float32)
        logits = logits + bg_ref[0]
        lane = lax.broadcasted_iota(jnp.int32, (TOK, E), 1)
        m1 = jnp.max(logits, axis=1, keepdims=True)
        i1 = jnp.min(jnp.where(logits == m1, lane, E), axis=1, keepdims=True)
        l2 = jnp.where(lane == i1, -jnp.inf, logits)
        m2 = jnp.max(l2, axis=1, keepdims=True)
        i2 = jnp.min(jnp.where(l2 == m2, lane, E), axis=1, keepdims=True)
        t = jnp.exp(m2 - m1)
        wa = 1.0 / (1.0 + t)
        wb = 1.0 - wa
        mw_ref[...] = jnp.where(lane == i1, wa, 0.0) + jnp.where(
            lane == i2, wb, 0.0)

    @pl.when(e < E)
    def _():
        w1f = w1_ref[0, :D, :] + w1_ref[0, D:, :]
        h = jnp.maximum(
            jnp.dot(x_ref[...], w1f, preferred_element_type=jnp.float32)
            + b1_ref[0, 0], 0.0)

        @pl.when(e % 2 == 0)
        def _():
            ha_ref[...] = h

        @pl.when(e % 2 == 1)
        def _():
            hb_ref[...] = h

    @pl.when(e > 0)
    def _():
        ep = e - 1
        mw = jnp.sum(
            mw_ref[...]
            * (lax.broadcasted_iota(jnp.int32, (TOK, E), 1) == ep).astype(
                jnp.float32),
            axis=1, keepdims=True)

        def consume(h_ref):
            y = jnp.dot(h_ref[...], w2_ref[0],
                        preferred_element_type=jnp.float32)
            contrib = mw * (y + b2_ref[0, 0])

            @pl.when(ep == 0)
            def _():
                out_ref[...] = contrib

            @pl.when(ep > 0)
            def _():
                out_ref[...] += contrib

        @pl.when(ep % 2 == 0)
        def _():
            consume(ha_ref)

        @pl.when(ep % 2 == 1)
        def _():
            consume(hb_ref)


def kernel(inputs, Wg, bg, W1, b1, W2, b2):
    bg2 = bg.reshape(1, E)
    b1r = b1.reshape(E, 1, DFF)
    b2r = b2.reshape(E, 1, OUT)
    out = pl.pallas_call(
        _moe_body,
        grid=(E + 1,),
        in_specs=[
            pl.BlockSpec((TOK, D), lambda e: (0, 0)),
            pl.BlockSpec((D, E), lambda e: (0, 0)),
            pl.BlockSpec((1, E), lambda e: (0, 0)),
            pl.BlockSpec((1, 2 * D, DFF),
                         lambda e: (jnp.minimum(e, E - 1), 0, 0)),
            pl.BlockSpec((1, 1, DFF), lambda e: (jnp.minimum(e, E - 1), 0, 0)),
            pl.BlockSpec((1, DFF, OUT),
                         lambda e: (jnp.maximum(e - 1, 0), 0, 0)),
            pl.BlockSpec((1, 1, OUT), lambda e: (jnp.maximum(e - 1, 0), 0, 0)),
        ],
        out_specs=pl.BlockSpec((TOK, OUT), lambda e: (0, 0)),
        out_shape=jax.ShapeDtypeStruct((TOK, OUT), jnp.float32),
        scratch_shapes=[
            pltpu.VMEM((TOK, DFF), jnp.float32),
            pltpu.VMEM((TOK, DFF), jnp.float32),
            pltpu.VMEM((TOK, E), jnp.float32),
        ],
        compiler_params=pltpu.CompilerParams(
            dimension_semantics=("arbitrary",),
        ),
    )(inputs, Wg, bg2, W1, b1r, W2, b2r)
    return out


kernel = jax.jit(kernel)
